# baseline (speedup 1.0000x reference)
import functools

import jax
import jax.numpy as jnp
from jax import lax
from jax.experimental import pallas as pl
from jax.experimental.pallas import tpu as pltpu
from jax.experimental.pallas import tpu_sc as plsc

BATCH = 16384
HIST = 50
D = 32
VOCAB_SZ = 1000000

NC = 2
NS = 16
NW = NC * NS
BPW = BATCH // NW
CHUNK = 128
NBLK = BPW // CHUNK
NUNIT = NBLK * HIST
NBUF = 4
NR = NUNIT // NBUF
TP = 129

NCB = VOCAB_SZ // CHUNK
CB_TAIL = VOCAB_SZ - NCB * CHUNK
NFULL = NCB // NW
NREM = NCB - NFULL * NW
TS = 33
DB = 2

_mesh = plsc.VectorSubcoreMesh(core_axis_name="c", subcore_axis_name="s")


@functools.partial(
    pl.kernel,
    mesh=_mesh,
    out_type=jax.ShapeDtypeStruct((VOCAB_SZ * D,), jnp.float32),
    scratch_types=[
        pltpu.VMEM((DB * 4 * 8, CHUNK), jnp.float32),
        pltpu.VMEM((DB * CHUNK * TS,), jnp.float32),
        pltpu.VMEM((DB * CHUNK * D,), jnp.float32),
        pltpu.SemaphoreType.DMA,
        pltpu.SemaphoreType.DMA,
    ],
    compiler_params=pltpu.CompilerParams(use_tc_tiling_on_sc=True,
                                         needs_layout_passes=False),
)
def _detile(lut_t, tail_lin, tab_hbm, stage_v, tbuf_v, cbuf_v, sem_i, sem_o):
    wid = lax.axis_index("s") * NC + lax.axis_index("c")
    iota = lax.iota(jnp.int32, 16)
    iota_ts = iota * TS

    def block_in(c, b):
        copies = []
        for r in range(4):
            cp = pltpu.make_async_copy(
                lut_t.at[pl.ds(8 * r, 8), pl.ds(c * CHUNK, CHUNK)],
                stage_v.at[pl.ds((b * 4 + r) * 8, 8), pl.ds(0, CHUNK)],
                sem_i)
            cp.start()
            copies.append(cp)
        return copies

    def block_compute(b, nv):
        tb = tbuf_v.at[pl.ds(b * CHUNK * TS, CHUNK * TS)]
        cb = cbuf_v.at[pl.ds(b * CHUNK * D, CHUNK * D)]
        for r in range(4):
            for s in range(8):
                f = 8 * r + s
                row = (b * 4 + r) * 8 + s
                vals = [stage_v[row, pl.ds(16 * i, 16)]
                        for i in range(nv // 16)]
                for i in range(nv // 16):
                    plsc.store_scatter(tb, [iota_ts + (16 * i * TS + f)],
                                       vals[i])
        for i in range(nv // 8):
            gs = []
            for j in range(8):
                v = 8 * i + j
                gs.append((plsc.load_gather(tb, [iota + (TS * v)]),
                           plsc.load_gather(tb, [iota + (TS * v + 16)])))
            for j in range(8):
                v = 8 * i + j
                ga, gb = gs[j]
                cb[pl.ds(D * v, 16)] = ga
                cb[pl.ds(D * v + 16, 16)] = gb

    def block_out(c, b, nv):
        oc = pltpu.make_async_copy(
            cbuf_v.at[pl.ds(b * CHUNK * D, nv * D)],
            tab_hbm.at[pl.ds(c * CHUNK * D, nv * D)], sem_o)
        oc.start()
        return oc

    def round_body(t, carry):
        copies = [block_in((DB * t + b) * NW + wid, b) for b in range(DB)]
        outs = []
        for b in range(DB):
            for cp in copies[b]:
                cp.wait()
            block_compute(b, CHUNK)
            outs.append(block_out((DB * t + b) * NW + wid, b, CHUNK))
        for oc in outs:
            oc.wait()
        return carry

    lax.fori_loop(0, NFULL // DB, round_body, 0)

    @pl.when(wid < NREM)
    def _rem():
        c = NFULL * NW + wid
        for cp in block_in(c, 0):
            cp.wait()
        block_compute(0, CHUNK)
        block_out(c, 0, CHUNK).wait()

    @pl.when(wid == NREM)
    def _tail():
        pltpu.sync_copy(tail_lin,
                        tab_hbm.at[pl.ds(NCB * CHUNK * D, CB_TAIL * D)])


@functools.partial(
    pl.kernel,
    mesh=_mesh,
    out_type=jax.ShapeDtypeStruct((HIST, D // 8, BATCH // CHUNK, 8, CHUNK),
                                  jnp.float32),
    scratch_types=[
        pltpu.VMEM((HIST, BPW), jnp.int32),
        pltpu.VMEM((NBUF, CHUNK, D), jnp.float32),
        pltpu.VMEM((NBUF, D, TP), jnp.float32),
        pltpu.SemaphoreType.DMA,
        pltpu.SemaphoreType.DMA,
    ],
    compiler_params=pltpu.CompilerParams(use_tc_tiling_on_sc=False,
                                         needs_layout_passes=False),
)
def _emb_lookup(xt_hbm, table_hbm, out_hbm, xloc, rows_v, tp_v, sem_g, sem_o):
    wid = lax.axis_index("s") * NC + lax.axis_index("c")
    b0w = wid * BPW
    pltpu.sync_copy(xt_hbm.at[:, pl.ds(b0w, BPW)], xloc)

    iota = lax.iota(jnp.int32, 16)
    row_lo = iota
    row_hi = iota + 16
    zerov = iota * 0

    def round_body(g, carry):
        gathers = []
        for b in range(NBUF):
            u = g * NBUF + b
            blk = u // HIST
            h = u % HIST
            c = pltpu.make_async_copy(
                table_hbm.at[xloc.at[h, pl.ds(blk * CHUNK, CHUNK)]],
                rows_v.at[b], sem_g)
            c.start()
            gathers.append(c)
        outs = []
        for b in range(NBUF):
            u = g * NBUF + b
            blk = u // HIST
            h = u % HIST
            gathers[b].wait()
            gbuf = rows_v.at[b]
            tbuf = tp_v.at[b]

            def tp_body(i, _, gbuf=gbuf, tbuf=tbuf):
                l0 = i * 8
                colv = zerov + l0
                vals = []
                for j in range(8):
                    vals.append((gbuf[l0 + j, pl.ds(0, 16)],
                                 gbuf[l0 + j, pl.ds(16, 16)]))
                for j in range(8):
                    lo, hi = vals[j]
                    col = colv + j
                    plsc.store_scatter(tbuf, [row_lo, col], lo)
                    plsc.store_scatter(tbuf, [row_hi, col], hi)
                return _

            lax.fori_loop(0, CHUNK // 8, tp_body, 0)
            for r in range(D // 8):
                oc = pltpu.make_async_copy(
                    tp_v.at[b, pl.ds(r * 8, 8), pl.ds(0, CHUNK)],
                    out_hbm.at[h, r, wid * NBLK + blk], sem_o)
                oc.start()
                outs.append(oc)
        for oc in outs:
            oc.wait()
        return carry

    lax.fori_loop(0, NR, round_body, 0)


def kernel(x, lut):
    xt = jnp.transpose(jnp.squeeze(x, axis=-1), (1, 0))
    tail = lut[NCB * CHUNK:].reshape(-1)
    tab = _detile(lut.T, tail).reshape(VOCAB_SZ, D)
    o5 = _emb_lookup(xt, tab)
    return jnp.transpose(o5, (2, 4, 0, 1, 3)).reshape(BATCH, HIST, D)


# speedup vs baseline: 1.9112x; 1.9112x over previous
import functools

import jax
import jax.numpy as jnp
from jax import lax
from jax.experimental import pallas as pl
from jax.experimental.pallas import tpu as pltpu
from jax.experimental.pallas import tpu_sc as plsc

BATCH = 16384
HIST = 50
D = 32
VOCAB_SZ = 1000000

NC = 2
NS = 16
NW = NC * NS
BPW = BATCH // NW
CHUNK = 128
NBLK = BPW // CHUNK
NUNIT = NBLK * HIST
NBUF = 4
NR = NUNIT // NBUF
TP = 129

NCB = VOCAB_SZ // CHUNK
CB_TAIL = VOCAB_SZ - NCB * CHUNK
NFULL = NCB // NW
NREM = NCB - NFULL * NW
TS = 33
DB = 2

_mesh = plsc.VectorSubcoreMesh(core_axis_name="c", subcore_axis_name="s")


@functools.partial(
    pl.kernel,
    mesh=_mesh,
    out_type=jax.ShapeDtypeStruct((VOCAB_SZ * D,), jnp.float32),
    scratch_types=[
        pltpu.VMEM((DB * 4 * 8, CHUNK), jnp.float32),
        pltpu.VMEM((DB * CHUNK * TS,), jnp.float32),
        pltpu.VMEM((DB * CHUNK * D,), jnp.float32),
        pltpu.SemaphoreType.DMA,
        pltpu.SemaphoreType.DMA,
    ],
    compiler_params=pltpu.CompilerParams(use_tc_tiling_on_sc=True,
                                         needs_layout_passes=False),
)
def _detile(lut_t, tail_lin, tab_hbm, stage_v, tbuf_v, cbuf_v, sem_i, sem_o):
    wid = lax.axis_index("s") * NC + lax.axis_index("c")
    iota = lax.iota(jnp.int32, 16)
    iota_ts = iota * TS

    def block_in(c, b):
        copies = []
        for r in range(4):
            cp = pltpu.make_async_copy(
                lut_t.at[pl.ds(8 * r, 8), pl.ds(c * CHUNK, CHUNK)],
                stage_v.at[pl.ds((b * 4 + r) * 8, 8), pl.ds(0, CHUNK)],
                sem_i)
            cp.start()
            copies.append(cp)
        return copies

    def block_compute(b):
        tb = tbuf_v.at[pl.ds(b * CHUNK * TS, CHUNK * TS)]
        cb = cbuf_v.at[pl.ds(b * CHUNK * D, CHUNK * D)]
        rows0 = b * 32

        def scat_body(i, base):
            off = 16 * i
            for half in range(2):
                vals = []
                for k in range(16):
                    f = 16 * half + k
                    vals.append(stage_v[rows0 + f, pl.ds(off, 16)])
                for k in range(16):
                    f = 16 * half + k
                    plsc.store_scatter(tb, [base + f], vals[k])
            return base + (16 * TS)

        lax.fori_loop(0, CHUNK // 16, scat_body, iota_ts)

        def comp_body(i, idx0):
            gs = []
            for j in range(8):
                ia = idx0 + (TS * j)
                gs.append((plsc.load_gather(tb, [ia]),
                           plsc.load_gather(tb, [ia + 16])))
            ob = i * (8 * D)
            for j in range(8):
                ga, gb = gs[j]
                cb[pl.ds(ob + D * j, 16)] = ga
                cb[pl.ds(ob + D * j + 16, 16)] = gb
            return idx0 + (TS * 8)

        lax.fori_loop(0, CHUNK // 8, comp_body, iota)

    def block_out(c, b, nv):
        oc = pltpu.make_async_copy(
            cbuf_v.at[pl.ds(b * CHUNK * D, nv * D)],
            tab_hbm.at[pl.ds(c * CHUNK * D, nv * D)], sem_o)
        oc.start()
        return oc

    def round_body(t, carry):
        copies = [block_in((DB * t + b) * NW + wid, b) for b in range(DB)]
        outs = []
        for b in range(DB):
            for cp in copies[b]:
                cp.wait()
            block_compute(b)
            outs.append(block_out((DB * t + b) * NW + wid, b, CHUNK))
        for oc in outs:
            oc.wait()
        return carry

    lax.fori_loop(0, NFULL // DB, round_body, 0)

    @pl.when(wid < NREM)
    def _rem():
        c = NFULL * NW + wid
        for cp in block_in(c, 0):
            cp.wait()
        block_compute(0)
        block_out(c, 0, CHUNK).wait()

    @pl.when(wid == NREM)
    def _tail():
        pltpu.sync_copy(tail_lin,
                        tab_hbm.at[pl.ds(NCB * CHUNK * D, CB_TAIL * D)])


@functools.partial(
    pl.kernel,
    mesh=_mesh,
    out_type=jax.ShapeDtypeStruct((HIST, D // 8, BATCH // CHUNK, 8, CHUNK),
                                  jnp.float32),
    scratch_types=[
        pltpu.VMEM((HIST, BPW), jnp.int32),
        pltpu.VMEM((NBUF, CHUNK, D), jnp.float32),
        pltpu.VMEM((NBUF, D, TP), jnp.float32),
        pltpu.SemaphoreType.DMA,
        pltpu.SemaphoreType.DMA,
    ],
    compiler_params=pltpu.CompilerParams(use_tc_tiling_on_sc=False,
                                         needs_layout_passes=False),
)
def _emb_lookup(xt_hbm, table_hbm, out_hbm, xloc, rows_v, tp_v, sem_g, sem_o):
    wid = lax.axis_index("s") * NC + lax.axis_index("c")
    b0w = wid * BPW
    pltpu.sync_copy(xt_hbm.at[:, pl.ds(b0w, BPW)], xloc)

    iota = lax.iota(jnp.int32, 16)
    row_lo = iota
    row_hi = iota + 16
    zerov = iota * 0

    def round_body(g, carry):
        gathers = []
        for b in range(NBUF):
            u = g * NBUF + b
            blk = u // HIST
            h = u % HIST
            c = pltpu.make_async_copy(
                table_hbm.at[xloc.at[h, pl.ds(blk * CHUNK, CHUNK)]],
                rows_v.at[b], sem_g)
            c.start()
            gathers.append(c)
        outs = []
        for b in range(NBUF):
            u = g * NBUF + b
            blk = u // HIST
            h = u % HIST
            gathers[b].wait()
            gbuf = rows_v.at[b]
            tbuf = tp_v.at[b]

            def tp_body(i, _, gbuf=gbuf, tbuf=tbuf):
                l0 = i * 8
                colv = zerov + l0
                vals = []
                for j in range(8):
                    vals.append((gbuf[l0 + j, pl.ds(0, 16)],
                                 gbuf[l0 + j, pl.ds(16, 16)]))
                for j in range(8):
                    lo, hi = vals[j]
                    col = colv + j
                    plsc.store_scatter(tbuf, [row_lo, col], lo)
                    plsc.store_scatter(tbuf, [row_hi, col], hi)
                return _

            lax.fori_loop(0, CHUNK // 8, tp_body, 0)
            for r in range(D // 8):
                oc = pltpu.make_async_copy(
                    tp_v.at[b, pl.ds(r * 8, 8), pl.ds(0, CHUNK)],
                    out_hbm.at[h, r, wid * NBLK + blk], sem_o)
                oc.start()
                outs.append(oc)
        for oc in outs:
            oc.wait()
        return carry

    lax.fori_loop(0, NR, round_body, 0)


def kernel(x, lut):
    xt = jnp.transpose(jnp.squeeze(x, axis=-1), (1, 0))
    tail = lut[NCB * CHUNK:].reshape(-1)
    tab = _detile(lut.T, tail).reshape(VOCAB_SZ, D)
    o5 = _emb_lookup(xt, tab)
    return jnp.transpose(o5, (2, 4, 0, 1, 3)).reshape(BATCH, HIST, D)


# speedup vs baseline: 2.7581x; 1.4432x over previous
import functools

import jax
import jax.numpy as jnp
from jax import lax
from jax.experimental import pallas as pl
from jax.experimental.pallas import tpu as pltpu
from jax.experimental.pallas import tpu_sc as plsc

BATCH = 16384
HIST = 50
D = 32
VOCAB_SZ = 1000000

NC = 2
NS = 16
NW = NC * NS
BPW = BATCH // NW
CHUNK = 128
NBLK = BPW // CHUNK
NUNIT = NBLK * HIST
NBUF = 4
NR = NUNIT // NBUF
TP = 129

NCB = VOCAB_SZ // CHUNK
CB_TAIL = VOCAB_SZ - NCB * CHUNK
NFULL = NCB // NW
NREM = NCB - NFULL * NW
TS = 33
DB = 2

_mesh = plsc.VectorSubcoreMesh(core_axis_name="c", subcore_axis_name="s")


@functools.partial(
    pl.kernel,
    mesh=_mesh,
    out_type=jax.ShapeDtypeStruct((VOCAB_SZ * D,), jnp.float32),
    scratch_types=[
        pltpu.VMEM((DB * 4 * 8, CHUNK), jnp.float32),
        pltpu.VMEM((DB * CHUNK * TS,), jnp.float32),
        pltpu.VMEM((DB * CHUNK * D,), jnp.float32),
        pltpu.SemaphoreType.DMA,
        pltpu.SemaphoreType.DMA,
    ],
    compiler_params=pltpu.CompilerParams(use_tc_tiling_on_sc=True,
                                         needs_layout_passes=False),
)
def _detile(lut_t, tail_lin, tab_hbm, stage_v, tbuf_v, cbuf_v, sem_i, sem_o):
    wid = lax.axis_index("s") * NC + lax.axis_index("c")
    iota = lax.iota(jnp.int32, 16)
    iota_ts = iota * TS

    def block_in_desc(c, b):
        return [pltpu.make_async_copy(
            lut_t.at[pl.ds(8 * r, 8), pl.ds(c * CHUNK, CHUNK)],
            stage_v.at[pl.ds((b * 4 + r) * 8, 8), pl.ds(0, CHUNK)],
            sem_i) for r in range(4)]

    def block_compute(b):
        tb = tbuf_v.at[pl.ds(b * CHUNK * TS, CHUNK * TS)]
        cb = cbuf_v.at[pl.ds(b * CHUNK * D, CHUNK * D)]
        rows0 = b * 32

        def scat_body(i, base):
            off = 16 * i
            for half in range(2):
                vals = []
                for k in range(16):
                    f = 16 * half + k
                    vals.append(stage_v[rows0 + f, pl.ds(off, 16)])
                for k in range(16):
                    f = 16 * half + k
                    plsc.store_scatter(tb, [base + f], vals[k])
            return base + (16 * TS)

        lax.fori_loop(0, CHUNK // 16, scat_body, iota_ts)

        def comp_body(i, idx0):
            gs = []
            for j in range(8):
                ia = idx0 + (TS * j)
                gs.append((plsc.load_gather(tb, [ia]),
                           plsc.load_gather(tb, [ia + 16])))
            ob = i * (8 * D)
            for j in range(8):
                ga, gb = gs[j]
                cb[pl.ds(ob + D * j, 16)] = ga
                cb[pl.ds(ob + D * j + 16, 16)] = gb
            return idx0 + (TS * 8)

        lax.fori_loop(0, CHUNK // 8, comp_body, iota)

    def block_out(c, b, nv):
        oc = pltpu.make_async_copy(
            cbuf_v.at[pl.ds(b * CHUNK * D, nv * D)],
            tab_hbm.at[pl.ds(c * CHUNK * D, nv * D)], sem_o)
        oc.start()
        return oc

    NT = NFULL // DB

    def round_body(t, carry):
        for b in range(DB):
            for cp in block_in_desc((DB * t + b) * NW + wid, b):
                cp.wait()

            @pl.when(t > 0)
            def _drain_prev(b=b):
                pltpu.make_async_copy(
                    cbuf_v.at[pl.ds(b * CHUNK * D, CHUNK * D)],
                    tab_hbm.at[pl.ds(0, CHUNK * D)], sem_o).wait()

            block_compute(b)
            block_out((DB * t + b) * NW + wid, b, CHUNK)

            @pl.when(t < NT - 1)
            def _prefetch(t=t, b=b):
                for cp in block_in_desc((DB * (t + 1) + b) * NW + wid, b):
                    cp.start()
        return carry

    for b in range(DB):
        for cp in block_in_desc((DB * 0 + b) * NW + wid, b):
            cp.start()
    lax.fori_loop(0, NT, round_body, 0)
    for b in range(DB):
        pltpu.make_async_copy(
            cbuf_v.at[pl.ds(b * CHUNK * D, CHUNK * D)],
            tab_hbm.at[pl.ds(0, CHUNK * D)], sem_o).wait()

    @pl.when(wid < NREM)
    def _rem():
        c = NFULL * NW + wid
        descs = block_in_desc(c, 0)
        for cp in descs:
            cp.start()
        for cp in descs:
            cp.wait()
        block_compute(0)
        block_out(c, 0, CHUNK).wait()

    @pl.when(wid == NREM)
    def _tail():
        pltpu.sync_copy(tail_lin,
                        tab_hbm.at[pl.ds(NCB * CHUNK * D, CB_TAIL * D)])


@functools.partial(
    pl.kernel,
    mesh=_mesh,
    out_type=jax.ShapeDtypeStruct((HIST, D // 8, BATCH // CHUNK, 8, CHUNK),
                                  jnp.float32),
    scratch_types=[
        pltpu.VMEM((HIST, BPW), jnp.int32),
        pltpu.VMEM((NBUF, CHUNK, D), jnp.float32),
        pltpu.VMEM((NBUF, D, TP), jnp.float32),
        pltpu.SemaphoreType.DMA,
        pltpu.SemaphoreType.DMA,
    ],
    compiler_params=pltpu.CompilerParams(use_tc_tiling_on_sc=False,
                                         needs_layout_passes=False),
)
def _emb_lookup(xt_hbm, table_hbm, out_hbm, xloc, rows_v, tp_v, sem_g, sem_o):
    wid = lax.axis_index("s") * NC + lax.axis_index("c")
    b0w = wid * BPW
    pltpu.sync_copy(xt_hbm.at[:, pl.ds(b0w, BPW)], xloc)

    iota = lax.iota(jnp.int32, 16)
    row_lo = iota
    row_hi = iota + 16
    zerov = iota * 0

    def gather_desc(u, b):
        blk = u // HIST
        h = u % HIST
        return pltpu.make_async_copy(
            table_hbm.at[xloc.at[h, pl.ds(blk * CHUNK, CHUNK)]],
            rows_v.at[b], sem_g)

    def drain_out_one(b):
        pltpu.make_async_copy(
            tp_v.at[b, pl.ds(0, 8), pl.ds(0, CHUNK)],
            out_hbm.at[0, 0, 0], sem_o).wait()

    def round_body(g, carry):
        for b in range(NBUF):
            u = g * NBUF + b
            blk = u // HIST
            h = u % HIST
            gather_desc(u, b).wait()

            @pl.when(g > 0)
            def _drain_prev(b=b):
                for _ in range(D // 8):
                    drain_out_one(b)

            gbuf = rows_v.at[b]
            tbuf = tp_v.at[b]

            def tp_body(i, _, gbuf=gbuf, tbuf=tbuf):
                l0 = i * 8
                colv = zerov + l0
                vals = []
                for j in range(8):
                    vals.append((gbuf[l0 + j, pl.ds(0, 16)],
                                 gbuf[l0 + j, pl.ds(16, 16)]))
                for j in range(8):
                    lo, hi = vals[j]
                    col = colv + j
                    plsc.store_scatter(tbuf, [row_lo, col], lo)
                    plsc.store_scatter(tbuf, [row_hi, col], hi)
                return _

            lax.fori_loop(0, CHUNK // 8, tp_body, 0)
            for r in range(D // 8):
                pltpu.make_async_copy(
                    tp_v.at[b, pl.ds(r * 8, 8), pl.ds(0, CHUNK)],
                    out_hbm.at[h, r, wid * NBLK + blk], sem_o).start()

            @pl.when(g < NR - 1)
            def _prefetch(g=g, b=b):
                gather_desc((g + 1) * NBUF + b, b).start()
        return carry

    for b in range(NBUF):
        gather_desc(b, b).start()
    lax.fori_loop(0, NR, round_body, 0)
    for b in range(NBUF):
        for _ in range(D // 8):
            drain_out_one(b)


def kernel(x, lut):
    xt = jnp.transpose(jnp.squeeze(x, axis=-1), (1, 0))
    tail = lut[NCB * CHUNK:].reshape(-1)
    tab = _detile(lut.T, tail).reshape(VOCAB_SZ, D)
    o5 = _emb_lookup(xt, tab)
    return jnp.transpose(o5, (2, 4, 0, 1, 3)).reshape(BATCH, HIST, D)


# speedup vs baseline: 3.2450x; 1.1765x over previous
import functools

import jax
import jax.numpy as jnp
from jax import lax
from jax.experimental import pallas as pl
from jax.experimental.pallas import tpu as pltpu
from jax.experimental.pallas import tpu_sc as plsc

BATCH = 16384
HIST = 50
D = 32
VOCAB_SZ = 1000000

NC = 2
NS = 16
NW = NC * NS
BPW = BATCH // NW
CHUNK = 128
NBLK = BPW // CHUNK
NUNIT = NBLK * HIST
NBUF = 4
NR = NUNIT // NBUF
TP = 129

NCB = VOCAB_SZ // CHUNK
CB_TAIL = VOCAB_SZ - NCB * CHUNK
NFULL = NCB // NW
NREM = NCB - NFULL * NW
TS = 33
DB = 4

_mesh = plsc.VectorSubcoreMesh(core_axis_name="c", subcore_axis_name="s")


@functools.partial(
    pl.kernel,
    mesh=_mesh,
    out_type=jax.ShapeDtypeStruct((VOCAB_SZ * D,), jnp.float32),
    scratch_types=[
        pltpu.VMEM((DB * 4 * 8, CHUNK), jnp.float32),
        pltpu.VMEM((DB * CHUNK * TS,), jnp.float32),
        pltpu.VMEM((DB * CHUNK * D,), jnp.float32),
        pltpu.SemaphoreType.DMA,
        pltpu.SemaphoreType.DMA,
    ],
    compiler_params=pltpu.CompilerParams(use_tc_tiling_on_sc=True,
                                         needs_layout_passes=False),
)
def _detile(lut_t, tail_lin, tab_hbm, stage_v, tbuf_v, cbuf_v, sem_i, sem_o):
    wid = lax.axis_index("s") * NC + lax.axis_index("c")
    iota = lax.iota(jnp.int32, 16)
    iota_ts = iota * TS

    def block_in_desc(c, b):
        return [pltpu.make_async_copy(
            lut_t.at[pl.ds(8 * r, 8), pl.ds(c * CHUNK, CHUNK)],
            stage_v.at[pl.ds((b * 4 + r) * 8, 8), pl.ds(0, CHUNK)],
            sem_i) for r in range(4)]

    def block_compute(b):
        tb = tbuf_v.at[pl.ds(b * CHUNK * TS, CHUNK * TS)]
        cb = cbuf_v.at[pl.ds(b * CHUNK * D, CHUNK * D)]
        rows0 = b * 32

        def scat_body(i, base):
            off = 16 * i
            for half in range(2):
                vals = []
                for k in range(16):
                    f = 16 * half + k
                    vals.append(stage_v[rows0 + f, pl.ds(off, 16)])
                for k in range(16):
                    f = 16 * half + k
                    plsc.store_scatter(tb, [base + f], vals[k])
            return base + (16 * TS)

        lax.fori_loop(0, CHUNK // 16, scat_body, iota_ts)

        def comp_body(i, idx0):
            gs = []
            for j in range(8):
                ia = idx0 + (TS * j)
                gs.append((plsc.load_gather(tb, [ia]),
                           plsc.load_gather(tb, [ia + 16])))
            ob = i * (8 * D)
            for j in range(8):
                ga, gb = gs[j]
                cb[pl.ds(ob + D * j, 16)] = ga
                cb[pl.ds(ob + D * j + 16, 16)] = gb
            return idx0 + (TS * 8)

        lax.fori_loop(0, CHUNK // 8, comp_body, iota)

    def block_out(c, b, nv):
        oc = pltpu.make_async_copy(
            cbuf_v.at[pl.ds(b * CHUNK * D, nv * D)],
            tab_hbm.at[pl.ds(c * CHUNK * D, nv * D)], sem_o)
        oc.start()
        return oc

    NT = NFULL // DB

    def round_body(t, carry):
        for b in range(DB):
            for cp in block_in_desc((DB * t + b) * NW + wid, b):
                cp.wait()

            @pl.when(t > 0)
            def _drain_prev(b=b):
                pltpu.make_async_copy(
                    cbuf_v.at[pl.ds(b * CHUNK * D, CHUNK * D)],
                    tab_hbm.at[pl.ds(0, CHUNK * D)], sem_o).wait()

            block_compute(b)
            block_out((DB * t + b) * NW + wid, b, CHUNK)

            @pl.when(t < NT - 1)
            def _prefetch(t=t, b=b):
                for cp in block_in_desc((DB * (t + 1) + b) * NW + wid, b):
                    cp.start()
        return carry

    for b in range(DB):
        for cp in block_in_desc((DB * 0 + b) * NW + wid, b):
            cp.start()
    lax.fori_loop(0, NT, round_body, 0)
    for b in range(DB):
        pltpu.make_async_copy(
            cbuf_v.at[pl.ds(b * CHUNK * D, CHUNK * D)],
            tab_hbm.at[pl.ds(0, CHUNK * D)], sem_o).wait()

    @pl.when(wid < NREM)
    def _rem():
        c = NFULL * NW + wid
        descs = block_in_desc(c, 0)
        for cp in descs:
            cp.start()
        for cp in descs:
            cp.wait()
        block_compute(0)
        block_out(c, 0, CHUNK).wait()

    @pl.when(wid == NREM)
    def _tail():
        pltpu.sync_copy(tail_lin,
                        tab_hbm.at[pl.ds(NCB * CHUNK * D, CB_TAIL * D)])


@functools.partial(
    pl.kernel,
    mesh=_mesh,
    out_type=jax.ShapeDtypeStruct((HIST, D // 8, BATCH // CHUNK, 8, CHUNK),
                                  jnp.float32),
    scratch_types=[
        pltpu.VMEM((HIST, BPW), jnp.int32),
        pltpu.VMEM((NBUF, CHUNK, D), jnp.float32),
        pltpu.VMEM((NBUF, D, TP), jnp.float32),
        pltpu.SemaphoreType.DMA,
        pltpu.SemaphoreType.DMA,
    ],
    compiler_params=pltpu.CompilerParams(use_tc_tiling_on_sc=False,
                                         needs_layout_passes=False),
)
def _emb_lookup(xt_hbm, table_hbm, out_hbm, xloc, rows_v, tp_v, sem_g, sem_o):
    wid = lax.axis_index("s") * NC + lax.axis_index("c")
    b0w = wid * BPW
    pltpu.sync_copy(xt_hbm.at[:, pl.ds(b0w, BPW)], xloc)

    iota = lax.iota(jnp.int32, 16)
    row_lo = iota
    row_hi = iota + 16
    zerov = iota * 0

    def gather_desc(u, b):
        blk = u // HIST
        h = u % HIST
        return pltpu.make_async_copy(
            table_hbm.at[xloc.at[h, pl.ds(blk * CHUNK, CHUNK)]],
            rows_v.at[b], sem_g)

    def drain_out_one(b):
        pltpu.make_async_copy(
            tp_v.at[b, pl.ds(0, 8), pl.ds(0, CHUNK)],
            out_hbm.at[0, 0, 0], sem_o).wait()

    def round_body(g, carry):
        for b in range(NBUF):
            u = g * NBUF + b
            blk = u // HIST
            h = u % HIST
            gather_desc(u, b).wait()

            @pl.when(g > 0)
            def _drain_prev(b=b):
                for _ in range(D // 8):
                    drain_out_one(b)

            gbuf = rows_v.at[b]
            tbuf = tp_v.at[b]

            def tp_body(i, _, gbuf=gbuf, tbuf=tbuf):
                l0 = i * 8
                colv = zerov + l0
                vals = []
                for j in range(8):
                    vals.append((gbuf[l0 + j, pl.ds(0, 16)],
                                 gbuf[l0 + j, pl.ds(16, 16)]))
                for j in range(8):
                    lo, hi = vals[j]
                    col = colv + j
                    plsc.store_scatter(tbuf, [row_lo, col], lo)
                    plsc.store_scatter(tbuf, [row_hi, col], hi)
                return _

            lax.fori_loop(0, CHUNK // 8, tp_body, 0)
            for r in range(D // 8):
                pltpu.make_async_copy(
                    tp_v.at[b, pl.ds(r * 8, 8), pl.ds(0, CHUNK)],
                    out_hbm.at[h, r, wid * NBLK + blk], sem_o).start()

            @pl.when(g < NR - 1)
            def _prefetch(g=g, b=b):
                gather_desc((g + 1) * NBUF + b, b).start()
        return carry

    for b in range(NBUF):
        gather_desc(b, b).start()
    lax.fori_loop(0, NR, round_body, 0)
    for b in range(NBUF):
        for _ in range(D // 8):
            drain_out_one(b)


def kernel(x, lut):
    xt = jnp.transpose(jnp.squeeze(x, axis=-1), (1, 0))
    tail = lut[NCB * CHUNK:].reshape(-1)
    tab = _detile(lut.T, tail).reshape(VOCAB_SZ, D)
    o5 = _emb_lookup(xt, tab)
    return jnp.transpose(o5, (2, 4, 0, 1, 3)).reshape(BATCH, HIST, D)


# speedup vs baseline: 3.3323x; 1.0269x over previous
import functools

import jax
import jax.numpy as jnp
from jax import lax
from jax.experimental import pallas as pl
from jax.experimental.pallas import tpu as pltpu
from jax.experimental.pallas import tpu_sc as plsc

BATCH = 16384
HIST = 50
D = 32
VOCAB_SZ = 1000000

NC = 2
NS = 16
NW = NC * NS
BPW = BATCH // NW
CHUNK = 128
NBLK = BPW // CHUNK
NUNIT = NBLK * HIST
NBUF = 8
NR = NUNIT // NBUF
TP = 129

NCB = VOCAB_SZ // CHUNK
CB_TAIL = VOCAB_SZ - NCB * CHUNK
NFULL = NCB // NW
NREM = NCB - NFULL * NW
TS = 33
DB = 4

_mesh = plsc.VectorSubcoreMesh(core_axis_name="c", subcore_axis_name="s")


@functools.partial(
    pl.kernel,
    mesh=_mesh,
    out_type=jax.ShapeDtypeStruct((VOCAB_SZ * D,), jnp.float32),
    scratch_types=[
        pltpu.VMEM((DB * 4 * 8, CHUNK), jnp.float32),
        pltpu.VMEM((DB * CHUNK * TS,), jnp.float32),
        pltpu.VMEM((DB * CHUNK * D,), jnp.float32),
        pltpu.SemaphoreType.DMA,
        pltpu.SemaphoreType.DMA,
    ],
    compiler_params=pltpu.CompilerParams(use_tc_tiling_on_sc=True,
                                         needs_layout_passes=False),
)
def _detile(lut_t, tail_lin, tab_hbm, stage_v, tbuf_v, cbuf_v, sem_i, sem_o):
    wid = lax.axis_index("s") * NC + lax.axis_index("c")
    iota = lax.iota(jnp.int32, 16)
    iota_ts = iota * TS

    def block_in_desc(c, b):
        return [pltpu.make_async_copy(
            lut_t.at[pl.ds(8 * r, 8), pl.ds(c * CHUNK, CHUNK)],
            stage_v.at[pl.ds((b * 4 + r) * 8, 8), pl.ds(0, CHUNK)],
            sem_i) for r in range(4)]

    def block_compute(b):
        tb = tbuf_v.at[pl.ds(b * CHUNK * TS, CHUNK * TS)]
        cb = cbuf_v.at[pl.ds(b * CHUNK * D, CHUNK * D)]
        rows0 = b * 32

        def scat_body(i, base):
            off = 16 * i
            for half in range(2):
                vals = []
                for k in range(16):
                    f = 16 * half + k
                    vals.append(stage_v[rows0 + f, pl.ds(off, 16)])
                for k in range(16):
                    f = 16 * half + k
                    plsc.store_scatter(tb, [base + f], vals[k])
            return base + (16 * TS)

        lax.fori_loop(0, CHUNK // 16, scat_body, iota_ts)

        def comp_body(i, idx0):
            gs = []
            for j in range(8):
                ia = idx0 + (TS * j)
                gs.append((plsc.load_gather(tb, [ia]),
                           plsc.load_gather(tb, [ia + 16])))
            ob = i * (8 * D)
            for j in range(8):
                ga, gb = gs[j]
                cb[pl.ds(ob + D * j, 16)] = ga
                cb[pl.ds(ob + D * j + 16, 16)] = gb
            return idx0 + (TS * 8)

        lax.fori_loop(0, CHUNK // 8, comp_body, iota)

    def block_out(c, b, nv):
        oc = pltpu.make_async_copy(
            cbuf_v.at[pl.ds(b * CHUNK * D, nv * D)],
            tab_hbm.at[pl.ds(c * CHUNK * D, nv * D)], sem_o)
        oc.start()
        return oc

    NT = NFULL // DB

    def round_body(t, carry):
        for b in range(DB):
            for cp in block_in_desc((DB * t + b) * NW + wid, b):
                cp.wait()

            @pl.when(t > 0)
            def _drain_prev(b=b):
                pltpu.make_async_copy(
                    cbuf_v.at[pl.ds(b * CHUNK * D, CHUNK * D)],
                    tab_hbm.at[pl.ds(0, CHUNK * D)], sem_o).wait()

            block_compute(b)
            block_out((DB * t + b) * NW + wid, b, CHUNK)

            @pl.when(t < NT - 1)
            def _prefetch(t=t, b=b):
                for cp in block_in_desc((DB * (t + 1) + b) * NW + wid, b):
                    cp.start()
        return carry

    for b in range(DB):
        for cp in block_in_desc((DB * 0 + b) * NW + wid, b):
            cp.start()
    lax.fori_loop(0, NT, round_body, 0)
    for b in range(DB):
        pltpu.make_async_copy(
            cbuf_v.at[pl.ds(b * CHUNK * D, CHUNK * D)],
            tab_hbm.at[pl.ds(0, CHUNK * D)], sem_o).wait()

    @pl.when(wid < NREM)
    def _rem():
        c = NFULL * NW + wid
        descs = block_in_desc(c, 0)
        for cp in descs:
            cp.start()
        for cp in descs:
            cp.wait()
        block_compute(0)
        block_out(c, 0, CHUNK).wait()

    @pl.when(wid == NREM)
    def _tail():
        pltpu.sync_copy(tail_lin,
                        tab_hbm.at[pl.ds(NCB * CHUNK * D, CB_TAIL * D)])


@functools.partial(
    pl.kernel,
    mesh=_mesh,
    out_type=jax.ShapeDtypeStruct((HIST, D // 8, BATCH // CHUNK, 8, CHUNK),
                                  jnp.float32),
    scratch_types=[
        pltpu.VMEM((HIST, BPW), jnp.int32),
        pltpu.VMEM((NBUF, CHUNK, D), jnp.float32),
        pltpu.VMEM((NBUF, D, TP), jnp.float32),
        pltpu.SemaphoreType.DMA,
        pltpu.SemaphoreType.DMA,
    ],
    compiler_params=pltpu.CompilerParams(use_tc_tiling_on_sc=False,
                                         needs_layout_passes=False),
)
def _emb_lookup(xt_hbm, table_hbm, out_hbm, xloc, rows_v, tp_v, sem_g, sem_o):
    wid = lax.axis_index("s") * NC + lax.axis_index("c")
    b0w = wid * BPW
    pltpu.sync_copy(xt_hbm.at[:, pl.ds(b0w, BPW)], xloc)

    iota = lax.iota(jnp.int32, 16)
    row_lo = iota
    row_hi = iota + 16
    zerov = iota * 0

    def gather_desc(u, b):
        blk = u // HIST
        h = u % HIST
        return pltpu.make_async_copy(
            table_hbm.at[xloc.at[h, pl.ds(blk * CHUNK, CHUNK)]],
            rows_v.at[b], sem_g)

    def drain_out_one(b):
        pltpu.make_async_copy(
            tp_v.at[b, pl.ds(0, 8), pl.ds(0, CHUNK)],
            out_hbm.at[0, 0, 0], sem_o).wait()

    def round_body(g, carry):
        for b in range(NBUF):
            u = g * NBUF + b
            blk = u // HIST
            h = u % HIST
            gather_desc(u, b).wait()

            @pl.when(g > 0)
            def _drain_prev(b=b):
                for _ in range(D // 8):
                    drain_out_one(b)

            gbuf = rows_v.at[b]
            tbuf = tp_v.at[b]

            def tp_body(i, _, gbuf=gbuf, tbuf=tbuf):
                l0 = i * 8
                colv = zerov + l0
                vals = []
                for j in range(8):
                    vals.append((gbuf[l0 + j, pl.ds(0, 16)],
                                 gbuf[l0 + j, pl.ds(16, 16)]))
                for j in range(8):
                    lo, hi = vals[j]
                    col = colv + j
                    plsc.store_scatter(tbuf, [row_lo, col], lo)
                    plsc.store_scatter(tbuf, [row_hi, col], hi)
                return _

            lax.fori_loop(0, CHUNK // 8, tp_body, 0)
            for r in range(D // 8):
                pltpu.make_async_copy(
                    tp_v.at[b, pl.ds(r * 8, 8), pl.ds(0, CHUNK)],
                    out_hbm.at[h, r, wid * NBLK + blk], sem_o).start()

            @pl.when(g < NR - 1)
            def _prefetch(g=g, b=b):
                gather_desc((g + 1) * NBUF + b, b).start()
        return carry

    for b in range(NBUF):
        gather_desc(b, b).start()
    lax.fori_loop(0, NR, round_body, 0)
    for b in range(NBUF):
        for _ in range(D // 8):
            drain_out_one(b)


def kernel(x, lut):
    xt = jnp.transpose(jnp.squeeze(x, axis=-1), (1, 0))
    tail = lut[NCB * CHUNK:].reshape(-1)
    tab = _detile(lut.T, tail).reshape(VOCAB_SZ, D)
    o5 = _emb_lookup(xt, tab)
    return jnp.transpose(o5, (2, 4, 0, 1, 3)).reshape(BATCH, HIST, D)
